# Initial kernel scaffold; baseline (speedup 1.0000x reference)
import jax
import jax.numpy as jnp
from jax.experimental import pallas as pl


def kernel(state, conv_w_0, conv_b_0, conv_w_1, conv_b_1, lin_w_0, lin_b_0, out_w, out_b):
    raise NotImplementedError("write your pallas kernel here")


# speedup vs baseline: 21.1839x; 21.1839x over previous
import jax
import jax.numpy as jnp
from jax.experimental import pallas as pl
from jax.experimental.pallas import tpu as pltpu

_LEAKY_SLOPE = 0.01
_EPS = 1e-5


def _leaky(x):
    return jnp.where(x > 0, x, _LEAKY_SLOPE * x)


def _fused_body(x_ref, w1_ref, b1_ref, w2_ref, b2_ref, wl_ref, bl_ref,
                wo_ref, bo_ref, o_ref):
    bt = x_ref.shape[-1]
    xb = x_ref[...]

    p1 = jnp.concatenate(
        [xb[:, dy * 20 + dx: dy * 20 + dx + 400, :]
         for dy in range(5) for dx in range(5)], axis=0)
    y1 = jnp.dot(w1_ref[...], p1.reshape(100, 400 * bt),
                 preferred_element_type=jnp.float32)
    y1 = _leaky(y1.reshape(32, 400, bt) + b1_ref[...].reshape(32, 1, 1))

    y1c = jnp.concatenate(
        [y1[:, i * 20: i * 20 + 16, :] for i in range(16)], axis=1)
    y1p = jnp.pad(y1c, ((0, 0), (0, 72), (0, 0)))

    w2 = w2_ref[...]
    acc = jnp.zeros((64, 256 * bt), jnp.float32)
    for dy in range(5):
        p2 = jnp.concatenate(
            [y1p[:, dy * 16 + dx: dy * 16 + dx + 256, :] for dx in range(5)],
            axis=0)
        acc = acc + jnp.dot(w2[:, dy * 160:(dy + 1) * 160],
                            p2.reshape(160, 256 * bt),
                            preferred_element_type=jnp.float32)
    y2 = _leaky(acc.reshape(64, 256, bt) + b2_ref[...].reshape(64, 1, 1))

    y2c = jnp.concatenate(
        [y2[:, i * 16: i * 16 + 12, :] for i in range(12)], axis=1)
    mu = jnp.mean(y2c, axis=1, keepdims=True)
    var = jnp.mean(jnp.square(y2c - mu), axis=1, keepdims=True)
    y2n = (y2c - mu) * jax.lax.rsqrt(var + _EPS)

    h = jnp.mean(y2n.reshape(64, 4, 36, bt), axis=2).reshape(256, bt)

    h = _leaky(jnp.dot(wl_ref[...], h, preferred_element_type=jnp.float32)
               + bl_ref[...])
    o = jnp.dot(wo_ref[...], h, preferred_element_type=jnp.float32) + bo_ref[...]
    o_ref[...] = o.astype(o_ref.dtype)


def kernel(state, conv_w_0, conv_b_0, conv_w_1, conv_b_1,
           lin_w_0, lin_b_0, out_w, out_b):
    B = state.shape[0]
    bt = 128
    x = jnp.transpose(state.reshape(B, 4, 400), (1, 2, 0))
    x = jnp.pad(x, ((0, 0), (0, 88), (0, 0)))
    w1 = jnp.transpose(conv_w_0, (3, 0, 1, 2)).reshape(32, 100)
    w2 = jnp.transpose(conv_w_1, (3, 0, 1, 2)).reshape(64, 800)
    b1 = conv_b_0.reshape(32, 1)
    b2 = conv_b_1.reshape(64, 1)
    wl = jnp.transpose(lin_w_0)
    bl = lin_b_0.reshape(256, 1)
    wo = jnp.pad(jnp.transpose(out_w), ((0, 4), (0, 0)))
    bo = jnp.pad(out_b.reshape(4, 1), ((0, 4), (0, 0)))

    out = pl.pallas_call(
        _fused_body,
        out_shape=jax.ShapeDtypeStruct((8, B), jnp.float32),
        grid=(B // bt,),
        in_specs=[
            pl.BlockSpec((4, 488, bt), lambda i: (0, 0, i)),
            pl.BlockSpec((32, 100), lambda i: (0, 0)),
            pl.BlockSpec((32, 1), lambda i: (0, 0)),
            pl.BlockSpec((64, 800), lambda i: (0, 0)),
            pl.BlockSpec((64, 1), lambda i: (0, 0)),
            pl.BlockSpec((256, 256), lambda i: (0, 0)),
            pl.BlockSpec((256, 1), lambda i: (0, 0)),
            pl.BlockSpec((8, 256), lambda i: (0, 0)),
            pl.BlockSpec((8, 1), lambda i: (0, 0)),
        ],
        out_specs=pl.BlockSpec((8, bt), lambda i: (0, i)),
        compiler_params=pltpu.CompilerParams(
            dimension_semantics=("parallel",)),
    )(x, w1, b1, w2, b2, wl, bl, wo, bo)
    return jnp.transpose(out[:4, :])


# speedup vs baseline: 21.5671x; 1.0181x over previous
import jax
import jax.numpy as jnp
from jax.experimental import pallas as pl
from jax.experimental.pallas import tpu as pltpu

_LEAKY_SLOPE = 0.01
_EPS = 1e-5


def _leaky(x):
    return jnp.where(x > 0, x, _LEAKY_SLOPE * x)


def _fused_body(x_ref, w1_ref, b1_ref, w2_ref, b2_ref, wl_ref, bl_ref,
                wo_ref, bo_ref, o_ref):
    bt = x_ref.shape[-1]
    xb = x_ref[...]

    p1 = jnp.concatenate(
        [xb[:, dy * 20 + dx: dy * 20 + dx + 400, :]
         for dy in range(5) for dx in range(5)], axis=0)
    y1 = jnp.dot(w1_ref[...], p1.reshape(100, 400 * bt),
                 preferred_element_type=jnp.float32)
    y1 = _leaky(y1.reshape(32, 400, bt) + b1_ref[...].reshape(32, 1, 1))
    y1 = y1.astype(jnp.bfloat16)

    y1c = jnp.concatenate(
        [y1[:, i * 20: i * 20 + 16, :] for i in range(16)], axis=1)
    y1p = jnp.pad(y1c, ((0, 0), (0, 72), (0, 0)))

    w2 = w2_ref[...]
    acc = jnp.zeros((64, 256 * bt), jnp.float32)
    for dy in range(5):
        p2 = jnp.concatenate(
            [y1p[:, dy * 16 + dx: dy * 16 + dx + 256, :] for dx in range(5)],
            axis=0)
        acc = acc + jnp.dot(w2[:, dy * 160:(dy + 1) * 160],
                            p2.reshape(160, 256 * bt),
                            preferred_element_type=jnp.float32)
    y2 = _leaky(acc.reshape(64, 256, bt) + b2_ref[...].reshape(64, 1, 1))

    y2c = jnp.concatenate(
        [y2[:, i * 16: i * 16 + 12, :] for i in range(12)], axis=1)
    mu = jnp.mean(y2c, axis=1, keepdims=True)
    var = jnp.mean(jnp.square(y2c - mu), axis=1, keepdims=True)
    y2n = (y2c - mu) * jax.lax.rsqrt(var + _EPS)

    h = jnp.mean(y2n.reshape(64, 4, 36, bt), axis=2).reshape(256, bt)

    h = _leaky(jnp.dot(wl_ref[...], h, preferred_element_type=jnp.float32)
               + bl_ref[...])
    o = jnp.dot(wo_ref[...], h, preferred_element_type=jnp.float32) + bo_ref[...]
    o_ref[...] = o.astype(o_ref.dtype)


def kernel(state, conv_w_0, conv_b_0, conv_w_1, conv_b_1,
           lin_w_0, lin_b_0, out_w, out_b):
    B = state.shape[0]
    bt = 128
    x = jnp.transpose(state.astype(jnp.bfloat16).reshape(B, 4, 400),
                      (1, 2, 0))
    x = jnp.pad(x, ((0, 0), (0, 88), (0, 0)))
    w1 = jnp.transpose(conv_w_0, (3, 0, 1, 2)).reshape(32, 100)
    w1 = w1.astype(jnp.bfloat16)
    w2 = jnp.transpose(conv_w_1, (3, 0, 1, 2)).reshape(64, 800)
    w2 = w2.astype(jnp.bfloat16)
    b1 = conv_b_0.reshape(32, 1)
    b2 = conv_b_1.reshape(64, 1)
    wl = jnp.transpose(lin_w_0)
    bl = lin_b_0.reshape(256, 1)
    wo = jnp.pad(jnp.transpose(out_w), ((0, 4), (0, 0)))
    bo = jnp.pad(out_b.reshape(4, 1), ((0, 4), (0, 0)))

    out = pl.pallas_call(
        _fused_body,
        out_shape=jax.ShapeDtypeStruct((8, B), jnp.float32),
        grid=(B // bt,),
        in_specs=[
            pl.BlockSpec((4, 488, bt), lambda i: (0, 0, i)),
            pl.BlockSpec((32, 100), lambda i: (0, 0)),
            pl.BlockSpec((32, 1), lambda i: (0, 0)),
            pl.BlockSpec((64, 800), lambda i: (0, 0)),
            pl.BlockSpec((64, 1), lambda i: (0, 0)),
            pl.BlockSpec((256, 256), lambda i: (0, 0)),
            pl.BlockSpec((256, 1), lambda i: (0, 0)),
            pl.BlockSpec((8, 256), lambda i: (0, 0)),
            pl.BlockSpec((8, 1), lambda i: (0, 0)),
        ],
        out_specs=pl.BlockSpec((8, bt), lambda i: (0, i)),
        compiler_params=pltpu.CompilerParams(
            dimension_semantics=("parallel",)),
    )(x, w1, b1, w2, b2, wl, bl, wo, bo)
    return jnp.transpose(out[:4, :])


# speedup vs baseline: 23.4000x; 1.0850x over previous
import jax
import jax.numpy as jnp
from jax.experimental import pallas as pl
from jax.experimental.pallas import tpu as pltpu

_LEAKY_SLOPE = 0.01
_EPS = 1e-5


def _leaky(x):
    return jnp.where(x > 0, x, _LEAKY_SLOPE * x)


def _fused_body(x_ref, w1_ref, b1_ref, w2_ref, b2_ref, wl_ref, bl_ref,
                wo_ref, bo_ref, o_ref):
    bt = x_ref.shape[-1]
    xb = x_ref[...]

    p1 = jnp.concatenate(
        [xb[:, dy * 20 + dx: dy * 20 + dx + 320, :]
         for dy in range(5) for dx in range(5)], axis=0)
    y1 = jnp.dot(w1_ref[...], p1.reshape(100, 320 * bt),
                 preferred_element_type=jnp.float32)
    y1 = _leaky(y1.reshape(32, 320, bt) + b1_ref[...].reshape(32, 1, 1))
    y1 = y1.astype(jnp.bfloat16)

    y1c = jnp.concatenate(
        [y1[:, i * 20: i * 20 + 16, :] for i in range(16)], axis=1)
    y1p = jnp.pad(y1c, ((0, 0), (0, 8), (0, 0)))

    w2 = w2_ref[...]
    acc = jnp.zeros((64, 192 * bt), jnp.float32)
    for dy in range(5):
        p2 = jnp.concatenate(
            [y1p[:, dy * 16 + dx: dy * 16 + dx + 192, :] for dx in range(5)],
            axis=0)
        acc = acc + jnp.dot(w2[:, dy * 160:(dy + 1) * 160],
                            p2.reshape(160, 192 * bt),
                            preferred_element_type=jnp.float32)
    y2 = _leaky(acc.reshape(64, 192, bt) + b2_ref[...].reshape(64, 1, 1))

    y2c = jnp.concatenate(
        [y2[:, i * 16: i * 16 + 12, :] for i in range(12)], axis=1)
    mu = jnp.mean(y2c, axis=1, keepdims=True)
    var = jnp.mean(jnp.square(y2c - mu), axis=1, keepdims=True)
    y2n = (y2c - mu) * jax.lax.rsqrt(var + _EPS)

    h = jnp.mean(y2n.reshape(64, 4, 36, bt), axis=2).reshape(256, bt)

    h = _leaky(jnp.dot(wl_ref[...], h, preferred_element_type=jnp.float32)
               + bl_ref[...])
    o = jnp.dot(wo_ref[...], h, preferred_element_type=jnp.float32) + bo_ref[...]
    o_ref[...] = o.astype(o_ref.dtype)


def kernel(state, conv_w_0, conv_b_0, conv_w_1, conv_b_1,
           lin_w_0, lin_b_0, out_w, out_b):
    B = state.shape[0]
    bt = 128
    x = jnp.transpose(state.astype(jnp.bfloat16).reshape(B, 4, 400),
                      (1, 2, 0))
    x = jnp.pad(x, ((0, 0), (0, 8), (0, 0)))
    w1 = jnp.transpose(conv_w_0, (3, 0, 1, 2)).reshape(32, 100)
    w1 = w1.astype(jnp.bfloat16)
    w2 = jnp.transpose(conv_w_1, (3, 0, 1, 2)).reshape(64, 800)
    w2 = w2.astype(jnp.bfloat16)
    b1 = conv_b_0.reshape(32, 1)
    b2 = conv_b_1.reshape(64, 1)
    wl = jnp.transpose(lin_w_0)
    bl = lin_b_0.reshape(256, 1)
    wo = jnp.pad(jnp.transpose(out_w), ((0, 4), (0, 0)))
    bo = jnp.pad(out_b.reshape(4, 1), ((0, 4), (0, 0)))

    out = pl.pallas_call(
        _fused_body,
        out_shape=jax.ShapeDtypeStruct((8, B), jnp.float32),
        grid=(B // bt,),
        in_specs=[
            pl.BlockSpec((4, 408, bt), lambda i: (0, 0, i)),
            pl.BlockSpec((32, 100), lambda i: (0, 0)),
            pl.BlockSpec((32, 1), lambda i: (0, 0)),
            pl.BlockSpec((64, 800), lambda i: (0, 0)),
            pl.BlockSpec((64, 1), lambda i: (0, 0)),
            pl.BlockSpec((256, 256), lambda i: (0, 0)),
            pl.BlockSpec((256, 1), lambda i: (0, 0)),
            pl.BlockSpec((8, 256), lambda i: (0, 0)),
            pl.BlockSpec((8, 1), lambda i: (0, 0)),
        ],
        out_specs=pl.BlockSpec((8, bt), lambda i: (0, i)),
        compiler_params=pltpu.CompilerParams(
            dimension_semantics=("parallel",)),
    )(x, w1, b1, w2, b2, wl, bl, wo, bo)
    return jnp.transpose(out[:4, :])
